# baseline (speedup 1.0000x reference)
import functools

import jax
import jax.numpy as jnp
from jax import lax
from jax.experimental import pallas as pl
from jax.experimental.pallas import tpu as pltpu
from jax.experimental.pallas import tpu_sc as plsc

N_NODES = 10000
N_EDGES = 320000
D = 128
N_GRAPHS = 64
BN_EPS = 1e-5

NC = 2
NS = 16
NW = NC * NS

CHUNK = 128
CPT = 80
EPT = CPT * CHUNK
REAL_EPT = N_EDGES // NW
E_PAD = EPT * NW

AGG_ROWS = 10240
RPT = AGG_ROWS // NS
RCHUNKS = RPT // CHUNK


def _sc_segment_sum_body(h_hbm, src_hbm, dst_hbm, zeros_hbm, out_hbm,
                         src_v, dst_v, rows_v, agg_sh, gsem0, gsem1):
    c = lax.axis_index("c")
    s = lax.axis_index("s")
    tile = c * NS + s

    pltpu.sync_copy(zeros_hbm, rows_v.at[0])
    for k in range(RCHUNKS):
        pltpu.sync_copy(rows_v.at[0],
                        agg_sh.at[pl.ds(s * RPT + k * CHUNK, CHUNK)])
    plsc.subcore_barrier()

    def body(j, _):
        i = j * 2
        pltpu.sync_copy(src_hbm.at[tile, i], src_v.at[0])
        pltpu.sync_copy(dst_hbm.at[tile, i], dst_v.at[0])
        pltpu.sync_copy(src_hbm.at[tile, i + 1], src_v.at[1])
        pltpu.sync_copy(dst_hbm.at[tile, i + 1], dst_v.at[1])
        d0 = pltpu.async_copy(h_hbm.at[src_v.at[0]], rows_v.at[0], gsem0)
        d1 = pltpu.async_copy(h_hbm.at[src_v.at[1]], rows_v.at[1], gsem1)
        d0.wait()
        pltpu.sync_copy(rows_v.at[0], agg_sh.at[dst_v.at[0]], add=True)
        d1.wait()
        pltpu.sync_copy(rows_v.at[1], agg_sh.at[dst_v.at[1]], add=True)
        return ()

    lax.fori_loop(0, CPT // 2, body, ())

    plsc.subcore_barrier()
    for k in range(RCHUNKS):
        r0 = s * RPT + k * CHUNK
        pltpu.sync_copy(agg_sh.at[pl.ds(r0, CHUNK)],
                        out_hbm.at[c].at[pl.ds(r0, CHUNK)])


@functools.cache
def _sc_segment_sum():
    return pl.kernel(
        _sc_segment_sum_body,
        out_type=jax.ShapeDtypeStruct((NC, AGG_ROWS, D), jnp.float32),
        mesh=plsc.VectorSubcoreMesh(core_axis_name="c", subcore_axis_name="s",
                                    num_cores=NC, num_subcores=NS),
        scratch_types=[
            pltpu.VMEM((2, CHUNK), jnp.int32),
            pltpu.VMEM((2, CHUNK), jnp.int32),
            pltpu.VMEM((2, CHUNK, D), jnp.float32),
            pltpu.VMEM_SHARED((AGG_ROWS, D), jnp.float32),
            pltpu.SemaphoreType.DMA,
            pltpu.SemaphoreType.DMA,
        ],
    )


def _tc_layer_body(h_ref, agg_ref, w1_ref, b1_ref, w2_ref, b2_ref,
                   gam_ref, bet_ref, o_ref):
    z = h_ref[...] + agg_ref[0, :N_NODES, :] + agg_ref[1, :N_NODES, :]
    z = jnp.dot(z, w1_ref[...], preferred_element_type=jnp.float32) + b1_ref[...]
    z = jnp.maximum(z, 0.0)
    z = jnp.dot(z, w2_ref[...], preferred_element_type=jnp.float32) + b2_ref[...]
    mean = jnp.mean(z, axis=0, keepdims=True)
    var = jnp.mean((z - mean) * (z - mean), axis=0, keepdims=True)
    z = (z - mean) * lax.rsqrt(var + BN_EPS) * gam_ref[...] + bet_ref[...]
    o_ref[...] = jnp.maximum(z, 0.0)


def _tc_layer(h, agg, w1, b1, w2, b2, gamma, beta):
    return pl.pallas_call(
        _tc_layer_body,
        out_shape=jax.ShapeDtypeStruct((N_NODES, D), jnp.float32),
    )(h, agg, w1, b1.reshape(1, D), w2, b2.reshape(1, D),
      gamma.reshape(1, D), beta.reshape(1, D))


def _tc_head_body(h_ref, b_ref, w1_ref, b1_ref, w2_ref, b2_ref, o_ref):
    gid = lax.broadcasted_iota(jnp.int32, (N_GRAPHS, N_NODES), 0)
    onehot = jnp.where(gid == b_ref[...], 1.0, 0.0)
    sums = jnp.dot(onehot, h_ref[...], preferred_element_type=jnp.float32)
    counts = jnp.sum(onehot, axis=1, keepdims=True)
    pooled = sums / jnp.maximum(counts, 1.0)
    t = jnp.dot(pooled, w1_ref[...], preferred_element_type=jnp.float32) + b1_ref[...]
    t = jnp.maximum(t, 0.0)
    o_ref[...] = jnp.dot(t, w2_ref[...], preferred_element_type=jnp.float32) + b2_ref[...]


def _tc_head(h, b, w1, b1, w2, b2):
    return pl.pallas_call(
        _tc_head_body,
        out_shape=jax.ShapeDtypeStruct((N_GRAPHS, D), jnp.float32),
    )(h, b.reshape(1, N_NODES), w1, b1.reshape(1, D), w2, b2.reshape(1, D))


def kernel(x, ei, b, params):
    ppt = EPT - REAL_EPT
    src = jnp.pad(ei[0].reshape(NW, REAL_EPT), ((0, 0), (0, ppt)))
    trash = N_NODES + (jnp.arange(ppt, dtype=jnp.int32)
                       % (AGG_ROWS - N_NODES))
    dst = jnp.concatenate(
        [ei[1].reshape(NW, REAL_EPT),
         jnp.broadcast_to(trash, (NW, ppt))], axis=1)
    src = src.reshape(NW, CPT, CHUNK)
    dst = dst.reshape(NW, CPT, CHUNK)
    zeros_blk = jnp.zeros((CHUNK, D), jnp.float32)

    h = x
    for layer in params["convs"]:
        agg = _sc_segment_sum()(h, src, dst, zeros_blk)
        h = _tc_layer(h, agg, layer["W1"], layer["b1"], layer["W2"],
                      layer["b2"], layer["gamma"], layer["beta"])
    hd = params["head"]
    return _tc_head(h, b, hd["W1"], hd["b1"], hd["W2"], hd["b2"])


# speedup vs baseline: 1.3514x; 1.3514x over previous
import functools

import jax
import jax.numpy as jnp
from jax import lax
from jax.experimental import pallas as pl
from jax.experimental.pallas import tpu as pltpu
from jax.experimental.pallas import tpu_sc as plsc

N_NODES = 10000
N_EDGES = 320000
D = 128
N_GRAPHS = 64
BN_EPS = 1e-5

NC = 2
NS = 16
NW = NC * NS

CHUNK = 128
CPT = 79
EPT = CPT * CHUNK
REAL_EPT = N_EDGES // NW
E_PAD = EPT * NW

AGG_ROWS = 10240
RPT = AGG_ROWS // NS
RCHUNKS = RPT // CHUNK


def _sc_segment_sum_body(h_hbm, src_hbm, dst_hbm, zeros_hbm, out_hbm,
                         src_v, dst_v, rows_v, zero_v, agg_sh, gsem):
    c = lax.axis_index("c")
    s = lax.axis_index("s")
    tile = c * NS + s

    pltpu.sync_copy(zeros_hbm, zero_v)
    for k in range(RCHUNKS):
        pltpu.sync_copy(zero_v, agg_sh.at[pl.ds(s * RPT + k * CHUNK, CHUNK)])
    plsc.subcore_barrier()

    def body(i, _):
        pltpu.sync_copy(src_hbm.at[tile, i], src_v)
        pltpu.sync_copy(dst_hbm.at[tile, i], dst_v)
        pltpu.async_copy(h_hbm.at[src_v], rows_v, gsem).wait()
        pltpu.sync_copy(rows_v, agg_sh.at[dst_v], add=True)
        return ()

    lax.fori_loop(0, CPT, body, ())

    plsc.subcore_barrier()
    for k in range(RCHUNKS):
        r0 = s * RPT + k * CHUNK
        pltpu.sync_copy(agg_sh.at[pl.ds(r0, CHUNK)],
                        out_hbm.at[c].at[pl.ds(r0, CHUNK)])


@functools.cache
def _sc_segment_sum():
    return pl.kernel(
        _sc_segment_sum_body,
        out_type=jax.ShapeDtypeStruct((NC, AGG_ROWS, D), jnp.float32),
        mesh=plsc.VectorSubcoreMesh(core_axis_name="c", subcore_axis_name="s",
                                    num_cores=NC, num_subcores=NS),
        scratch_types=[
            pltpu.VMEM((CHUNK,), jnp.int32),
            pltpu.VMEM((CHUNK,), jnp.int32),
            pltpu.VMEM((CHUNK, D), jnp.float32),
            pltpu.VMEM((CHUNK, D), jnp.float32),
            pltpu.VMEM_SHARED((AGG_ROWS, D), jnp.float32),
            pltpu.SemaphoreType.DMA,
        ],
    )


def _tc_layer_body(h_ref, agg_ref, w1_ref, b1_ref, w2_ref, b2_ref,
                   gam_ref, bet_ref, o_ref):
    z = h_ref[...] + agg_ref[0, :N_NODES, :] + agg_ref[1, :N_NODES, :]
    z = jnp.dot(z, w1_ref[...], preferred_element_type=jnp.float32) + b1_ref[...]
    z = jnp.maximum(z, 0.0)
    z = jnp.dot(z, w2_ref[...], preferred_element_type=jnp.float32) + b2_ref[...]
    mean = jnp.mean(z, axis=0, keepdims=True)
    var = jnp.mean((z - mean) * (z - mean), axis=0, keepdims=True)
    z = (z - mean) * lax.rsqrt(var + BN_EPS) * gam_ref[...] + bet_ref[...]
    o_ref[...] = jnp.maximum(z, 0.0)


def _tc_layer(h, agg, w1, b1, w2, b2, gamma, beta):
    return pl.pallas_call(
        _tc_layer_body,
        out_shape=jax.ShapeDtypeStruct((N_NODES, D), jnp.float32),
    )(h, agg, w1, b1.reshape(1, D), w2, b2.reshape(1, D),
      gamma.reshape(1, D), beta.reshape(1, D))


def _tc_head_body(h_ref, b_ref, w1_ref, b1_ref, w2_ref, b2_ref, o_ref):
    gid = lax.broadcasted_iota(jnp.int32, (N_GRAPHS, N_NODES), 0)
    onehot = jnp.where(gid == b_ref[...], 1.0, 0.0)
    sums = jnp.dot(onehot, h_ref[...], preferred_element_type=jnp.float32)
    counts = jnp.sum(onehot, axis=1, keepdims=True)
    pooled = sums / jnp.maximum(counts, 1.0)
    t = jnp.dot(pooled, w1_ref[...], preferred_element_type=jnp.float32) + b1_ref[...]
    t = jnp.maximum(t, 0.0)
    o_ref[...] = jnp.dot(t, w2_ref[...], preferred_element_type=jnp.float32) + b2_ref[...]


def _tc_head(h, b, w1, b1, w2, b2):
    return pl.pallas_call(
        _tc_head_body,
        out_shape=jax.ShapeDtypeStruct((N_GRAPHS, D), jnp.float32),
    )(h, b.reshape(1, N_NODES), w1, b1.reshape(1, D), w2, b2.reshape(1, D))


def kernel(x, ei, b, params):
    ppt = EPT - REAL_EPT
    src = jnp.pad(ei[0].reshape(NW, REAL_EPT), ((0, 0), (0, ppt)))
    trash = N_NODES + (jnp.arange(ppt, dtype=jnp.int32)
                       % (AGG_ROWS - N_NODES))
    dst = jnp.concatenate(
        [ei[1].reshape(NW, REAL_EPT),
         jnp.broadcast_to(trash, (NW, ppt))], axis=1)
    src = src.reshape(NW, CPT, CHUNK)
    dst = dst.reshape(NW, CPT, CHUNK)
    zeros_blk = jnp.zeros((CHUNK, D), jnp.float32)

    h = x
    for layer in params["convs"]:
        agg = _sc_segment_sum()(h, src, dst, zeros_blk)
        h = _tc_layer(h, agg, layer["W1"], layer["b1"], layer["W2"],
                      layer["b2"], layer["gamma"], layer["beta"])
    hd = params["head"]
    return _tc_head(h, b, hd["W1"], hd["b1"], hd["W2"], hd["b2"])
